# baseline (speedup 1.0000x reference)
import functools

import jax
import jax.numpy as jnp
from jax import lax
from jax.experimental import pallas as pl
from jax.experimental.pallas import tpu as pltpu
from jax.experimental.pallas import tpu_sc as plsc

_BATCH = 1024
_SEQ = 200
_N = _BATCH * _SEQ
_NC = 2
_NS = 16
_NW = _NC * _NS
_ROWS_W = _BATCH // _NW
_PER_W = _N // _NW
_CHUNK = 128
_NCHUNK = _PER_W // _CHUNK
_HVEC = _PER_W // 16
_EDIM = 64
_MDIM = 512
_MULT_A = 36313
_MULT_B = 27191


def _sc_body(tok_hbm, table_hbm, out_hbm, tok_v, idx_v, rows_v, sem):
    wid = lax.axis_index("s") * _NC + lax.axis_index("c")
    base = wid * _PER_W
    mod = table_hbm.shape[0] - 1

    def row_copy(r, _):
        pltpu.sync_copy(tok_hbm.at[wid * _ROWS_W + r],
                        tok_v.at[pl.ds(8 + r * _SEQ, _SEQ)])
        return 0

    lax.fori_loop(0, _ROWS_W, row_copy, 0)

    def hash_step(k, _):
        cur = tok_v[pl.ds(8 + k * 16, 16)]
        prev = tok_v[pl.ds(7 + k * 16, 16)]
        h = (_MULT_A * cur ^ _MULT_B * prev) % mod
        pos = k * 16 + lax.iota(jnp.int32, 16)
        idx_v[pl.ds(k * 16, 16)] = jnp.where(pos % _SEQ == 0, mod, h)
        return 0

    lax.fori_loop(0, _HVEC, hash_step, 0)

    def gather_step(c, _):
        pltpu.async_copy(
            table_hbm.at[idx_v.at[pl.ds(c * _CHUNK, _CHUNK)]], rows_v,
            sem).wait()
        pltpu.sync_copy(rows_v, out_hbm.at[pl.ds(base + c * _CHUNK, _CHUNK)])
        return 0

    lax.fori_loop(0, _NCHUNK, gather_step, 0)


@jax.jit
def _sc_hash_gather(tok2d, table):
    mesh = plsc.VectorSubcoreMesh(
        core_axis_name="c", subcore_axis_name="s", num_cores=_NC,
        num_subcores=_NS)
    f = pl.kernel(
        _sc_body,
        out_type=jax.ShapeDtypeStruct((_N, _EDIM), jnp.float32),
        mesh=mesh,
        scratch_types=[
            pltpu.VMEM((_PER_W + 8,), jnp.int32),
            pltpu.VMEM((_PER_W,), jnp.int32),
            pltpu.VMEM((_CHUNK, _EDIM), jnp.float32),
            pltpu.SemaphoreType.DMA,
        ],
        compiler_params=pltpu.CompilerParams(use_tc_tiling_on_sc=False),
    )
    return f(tok2d, table)


_RB = 1024


def _mm_body(h_ref, w_ref, o_ref):
    o_ref[...] = jnp.dot(h_ref[...], w_ref[...],
                         preferred_element_type=jnp.float32)


@jax.jit
def _tc_project(h, w):
    return pl.pallas_call(
        _mm_body,
        grid=(_N // _RB,),
        in_specs=[
            pl.BlockSpec((_RB, _EDIM), lambda i: (i, 0)),
            pl.BlockSpec((_EDIM, _MDIM), lambda i: (0, 0)),
        ],
        out_specs=pl.BlockSpec((_RB, _MDIM), lambda i: (i, 0)),
        out_shape=jax.ShapeDtypeStruct((_N, _MDIM), jnp.float32),
    )(h, w)


def kernel(token_ids, embed_weight, proj_weight, scale):
    gathered = _sc_hash_gather(token_ids, embed_weight)
    w = (proj_weight * scale).T
    out = _tc_project(gathered, w)
    return out.reshape(_BATCH, _SEQ, _MDIM)


# speedup vs baseline: 1.0990x; 1.0990x over previous
import functools

import jax
import jax.numpy as jnp
from jax import lax
from jax.experimental import pallas as pl
from jax.experimental.pallas import tpu as pltpu
from jax.experimental.pallas import tpu_sc as plsc

_BATCH = 1024
_SEQ = 200
_N = _BATCH * _SEQ
_NC = 2
_NS = 16
_NW = _NC * _NS
_ROWS_W = _BATCH // _NW
_PER_W = _N // _NW
_CHUNK = 128
_NCHUNK = _PER_W // _CHUNK
_HVEC = _PER_W // 16
_EDIM = 64
_PDIM = 128
_MDIM = 512
_MULT_A = 36313
_MULT_B = 27191
_MOD = 999999


def _repack_body(tok_hbm, out_hbm, tok8_v, flat_v):
    wid = lax.axis_index("s") * _NC + lax.axis_index("c")

    def grp(g, _):
        row0 = wid * _ROWS_W + g * 8
        pltpu.sync_copy(tok_hbm.at[pl.ds(row0, 8)], tok8_v)

        def row(r, _):
            rbase = (g * 8 + r) * _SEQ
            for c in tuple(range(0, 192, 16)) + (_SEQ - 16,):
                flat_v[pl.ds(rbase + c, 16)] = tok8_v[r, pl.ds(c, 16)]
            return 0

        lax.fori_loop(0, 8, row, 0)
        return 0

    lax.fori_loop(0, _ROWS_W // 8, grp, 0)
    pltpu.sync_copy(flat_v, out_hbm.at[pl.ds(wid * _PER_W, _PER_W)])


def _sc_body(tok_hbm, table_hbm, out_hbm, tok_v, idx_v, rows_v, sem):
    wid = lax.axis_index("s") * _NC + lax.axis_index("c")
    base = wid * _PER_W

    pltpu.sync_copy(tok_hbm.at[pl.ds(base, _PER_W)], tok_v.at[pl.ds(8, _PER_W)])

    def hash_step(k, _):
        cur = tok_v[pl.ds(8 + k * 16, 16)]
        prev = tok_v[pl.ds(7 + k * 16, 16)]
        h = (_MULT_A * cur ^ _MULT_B * prev) % _MOD
        pos = k * 16 + lax.iota(jnp.int32, 16)
        idx_v[pl.ds(k * 16, 16)] = jnp.where(pos % _SEQ == 0, _MOD, h)
        return 0

    lax.fori_loop(0, _HVEC, hash_step, 0)

    def gather_step(c, _):
        pltpu.async_copy(
            table_hbm.at[idx_v.at[pl.ds(c * _CHUNK, _CHUNK)]], rows_v,
            sem).wait()
        pltpu.sync_copy(
            rows_v,
            out_hbm.at[pl.ds(base + c * _CHUNK, _CHUNK), pl.ds(0, _EDIM)])
        return 0

    lax.fori_loop(0, _NCHUNK, gather_step, 0)


_MESH = dict(core_axis_name="c", subcore_axis_name="s", num_cores=_NC,
             num_subcores=_NS)


@jax.jit
def _sc_pipeline(tok2d, table):
    repack_k = pl.kernel(
        _repack_body,
        out_type=jax.ShapeDtypeStruct((_N,), jnp.int32),
        mesh=plsc.VectorSubcoreMesh(**_MESH),
        scratch_types=[
            pltpu.VMEM((8, _SEQ), jnp.int32),
            pltpu.VMEM((_PER_W,), jnp.int32),
        ],
        compiler_params=pltpu.CompilerParams(use_tc_tiling_on_sc=True),
    )
    tok_flat = repack_k(tok2d)

    gather_k = pl.kernel(
        _sc_body,
        out_type=jax.ShapeDtypeStruct((_N, _PDIM), jnp.float32),
        mesh=plsc.VectorSubcoreMesh(**_MESH),
        scratch_types=[
            pltpu.VMEM((_PER_W + 8,), jnp.int32),
            pltpu.VMEM((_PER_W,), jnp.int32),
            pltpu.VMEM((_CHUNK, _EDIM), jnp.float32),
            pltpu.SemaphoreType.DMA,
        ],
        compiler_params=pltpu.CompilerParams(use_tc_tiling_on_sc=False),
    )
    return gather_k(tok_flat, table)


_RB = 1024


def _mm_body(h_ref, w_ref, o_ref):
    o_ref[...] = jnp.dot(h_ref[:, :_EDIM], w_ref[...],
                         preferred_element_type=jnp.float32)


@jax.jit
def _tc_project(h, w):
    return pl.pallas_call(
        _mm_body,
        grid=(_N // _RB,),
        in_specs=[
            pl.BlockSpec((_RB, _PDIM), lambda i: (i, 0)),
            pl.BlockSpec((_EDIM, _MDIM), lambda i: (0, 0)),
        ],
        out_specs=pl.BlockSpec((_RB, _MDIM), lambda i: (i, 0)),
        out_shape=jax.ShapeDtypeStruct((_N, _MDIM), jnp.float32),
    )(h, w)


def kernel(token_ids, embed_weight, proj_weight, scale):
    gathered = _sc_pipeline(token_ids, embed_weight)
    w = (proj_weight * scale).T
    out = _tc_project(gathered, w)
    return out.reshape(_BATCH, _SEQ, _MDIM)
